# baseline (speedup 1.0000x reference)
import functools
import jax
import jax.numpy as jnp
from jax import lax
from jax.experimental import pallas as pl
from jax.experimental.pallas import tpu as pltpu
from jax.experimental.pallas import tpu_sc as plsc

BATCH = 4
SEQ = 2048
DM = 2048
NC = 2
NS = 16
NW = NC * NS
ROWS_PER_W = SEQ // NW
CHUNK = 8
N_CHUNKS = ROWS_PER_W // CHUNK
NB = 3


def _sc_body(x_hbm, p_hbm, o_hbm,
             xbuf0, xbuf1, xbuf2, pbuf0, pbuf1, shared,
             sx0, sx1, sx2, sp0, sp1, so0, so1, so2):
    wid = lax.axis_index("c") * NS + lax.axis_index("s")
    sid = lax.axis_index("s")
    row_base = wid * ROWS_PER_W

    def sh_view(s):
        return shared.at[pl.ds((sid * NB + s) * CHUNK, CHUNK), :]
    xbufs = (xbuf0, xbuf1, xbuf2)
    pbufs = (pbuf0, pbuf1)
    sxs = (sx0, sx1, sx2)
    sps = (sp0, sp1)
    sos = (so0, so1, so2)

    def x_src(c, b):
        return x_hbm.at[b, pl.ds(row_base + c * CHUNK, CHUNK), :]

    def o_dst(c, b):
        return o_hbm.at[b, pl.ds(row_base + c * CHUNK, CHUNK), :]

    def p_src(c):
        return p_hbm.at[pl.ds(row_base + c * CHUNK, CHUNK), :]

    items = [(c, b) for c in range(N_CHUNKS) for b in range(BATCH)]
    n = len(items)

    pltpu.make_async_copy(p_src(0), pbuf0, sp0).start()
    pltpu.make_async_copy(x_src(0, 0), xbuf0, sx0).start()

    for i, (c, b) in enumerate(items):
        s = i % NB
        ps = c % 2
        if b == 0:
            pltpu.make_async_copy(p_src(c), pbufs[ps], sps[ps]).wait()
            if c + 1 < N_CHUNKS:
                nps = (c + 1) % 2
                pltpu.make_async_copy(p_src(c + 1), pbufs[nps], sps[nps]).start()
        if i + 1 < n:
            ns = (i + 1) % NB
            if i >= NB - 1:
                pc, pb = items[i + 1 - NB]
                pltpu.make_async_copy(sh_view(ns), o_dst(pc, pb), sos[ns]).wait()
            nc, nb = items[i + 1]
            pltpu.make_async_copy(x_src(nc, nb), xbufs[ns], sxs[ns]).start()
        pltpu.make_async_copy(x_src(c, b), xbufs[s], sxs[s]).wait()

        xb, pb_ = xbufs[s], pbufs[ps]

        def vbody(j, xb=xb, pb_=pb_):
            for r in range(CHUNK):
                plsc.addupdate(xb.at[r, pl.ds(j, 16)], pb_[r, pl.ds(j, 16)])

        plsc.parallel_loop(0, DM, step=16, unroll=2)(vbody)

        pltpu.sync_copy(xbufs[s], sh_view(s))
        pltpu.make_async_copy(sh_view(s), o_dst(c, b), sos[s]).start()

    for i in range(n - NB, n):
        ce, be = items[i]
        pltpu.make_async_copy(sh_view(i % NB), o_dst(ce, be), sos[i % NB]).wait()


def kernel(inputs, pos_table):
    mesh = plsc.VectorSubcoreMesh(core_axis_name="c", subcore_axis_name="s")
    k = functools.partial(
        pl.kernel,
        mesh=mesh,
        out_type=jax.ShapeDtypeStruct((BATCH, SEQ, DM), jnp.float32),
        scratch_types=[
            pltpu.VMEM((CHUNK, DM), jnp.float32),
            pltpu.VMEM((CHUNK, DM), jnp.float32),
            pltpu.VMEM((CHUNK, DM), jnp.float32),
            pltpu.VMEM((CHUNK, DM), jnp.float32),
            pltpu.VMEM((CHUNK, DM), jnp.float32),
            pltpu.VMEM_SHARED((NS * NB * CHUNK, DM), jnp.float32),
            pltpu.SemaphoreType.DMA,
            pltpu.SemaphoreType.DMA,
            pltpu.SemaphoreType.DMA,
            pltpu.SemaphoreType.DMA,
            pltpu.SemaphoreType.DMA,
            pltpu.SemaphoreType.DMA,
            pltpu.SemaphoreType.DMA,
            pltpu.SemaphoreType.DMA,
        ],
    )(_sc_body)
    return k(inputs, pos_table)


# speedup vs baseline: 1.0976x; 1.0976x over previous
import functools
import jax
import jax.numpy as jnp
from jax import lax
from jax.experimental import pallas as pl
from jax.experimental.pallas import tpu as pltpu
from jax.experimental.pallas import tpu_sc as plsc

BATCH = 4
SEQ = 2048
DM = 2048
NC = 2
NS = 16
NW = NC * NS
ROWS_PER_W = SEQ // NW
CHUNK = 8
N_CHUNKS = ROWS_PER_W // CHUNK
NB = 3


def _sc_body(x_hbm, p_hbm, o_hbm,
             xbuf0, xbuf1, xbuf2, pbuf0, pbuf1,
             sx0, sx1, sx2, sp0, sp1, so0, so1, so2):
    wid = lax.axis_index("c") * NS + lax.axis_index("s")
    row_base = wid * ROWS_PER_W
    xbufs = (xbuf0, xbuf1, xbuf2)
    pbufs = (pbuf0, pbuf1)
    sxs = (sx0, sx1, sx2)
    sps = (sp0, sp1)
    sos = (so0, so1, so2)

    def x_src(c, b):
        return x_hbm.at[b, pl.ds(row_base + c * CHUNK, CHUNK), :]

    def o_dst(c, b):
        return o_hbm.at[b, pl.ds(row_base + c * CHUNK, CHUNK), :]

    def p_src(c):
        return p_hbm.at[pl.ds(row_base + c * CHUNK, CHUNK), :]

    items = [(c, b) for c in range(N_CHUNKS) for b in range(BATCH)]
    n = len(items)

    pltpu.make_async_copy(p_src(0), pbuf0, sp0).start()
    pltpu.make_async_copy(x_src(0, 0), xbuf0, sx0).start()

    for i, (c, b) in enumerate(items):
        s = i % NB
        ps = c % 2
        if b == 0:
            pltpu.make_async_copy(p_src(c), pbufs[ps], sps[ps]).wait()
            if c + 1 < N_CHUNKS:
                nps = (c + 1) % 2
                pltpu.make_async_copy(p_src(c + 1), pbufs[nps], sps[nps]).start()
        if i + 1 < n:
            ns = (i + 1) % NB
            if i >= NB - 1:
                pc, pb = items[i + 1 - NB]
                pltpu.make_async_copy(xbufs[ns], o_dst(pc, pb), sos[ns]).wait()
            nc, nb = items[i + 1]
            pltpu.make_async_copy(x_src(nc, nb), xbufs[ns], sxs[ns]).start()
        pltpu.make_async_copy(x_src(c, b), xbufs[s], sxs[s]).wait()

        xb, pb_ = xbufs[s], pbufs[ps]

        def vbody(j, xb=xb, pb_=pb_):
            for r in range(CHUNK):
                plsc.addupdate(xb.at[r, pl.ds(j, 16)], pb_[r, pl.ds(j, 16)])

        plsc.parallel_loop(0, DM, step=16, unroll=2)(vbody)

        pltpu.make_async_copy(xbufs[s], o_dst(c, b), sos[s]).start()

    for i in range(n - NB, n):
        ce, be = items[i]
        pltpu.make_async_copy(xbufs[i % NB], o_dst(ce, be), sos[i % NB]).wait()


def kernel(inputs, pos_table):
    mesh = plsc.VectorSubcoreMesh(core_axis_name="c", subcore_axis_name="s")
    k = functools.partial(
        pl.kernel,
        mesh=mesh,
        out_type=jax.ShapeDtypeStruct((BATCH, SEQ, DM), jnp.float32),
        scratch_types=[
            pltpu.VMEM((CHUNK, DM), jnp.float32),
            pltpu.VMEM((CHUNK, DM), jnp.float32),
            pltpu.VMEM((CHUNK, DM), jnp.float32),
            pltpu.VMEM((CHUNK, DM), jnp.float32),
            pltpu.VMEM((CHUNK, DM), jnp.float32),
            pltpu.SemaphoreType.DMA,
            pltpu.SemaphoreType.DMA,
            pltpu.SemaphoreType.DMA,
            pltpu.SemaphoreType.DMA,
            pltpu.SemaphoreType.DMA,
            pltpu.SemaphoreType.DMA,
            pltpu.SemaphoreType.DMA,
            pltpu.SemaphoreType.DMA,
        ],
    )(_sc_body)
    return k(inputs, pos_table)


# speedup vs baseline: 1.0981x; 1.0005x over previous
import functools
import jax
import jax.numpy as jnp
from jax import lax
from jax.experimental import pallas as pl
from jax.experimental.pallas import tpu as pltpu
from jax.experimental.pallas import tpu_sc as plsc

NC = 2
NS = 16
NB = 3
CHUNK = 8


def _make_body(batch, seq, dm):
    nw = NC * NS
    rows_per_w = seq // nw
    n_chunks = rows_per_w // CHUNK

    def body(x_hbm, p_hbm, o_hbm,
             xbuf0, xbuf1, xbuf2, pbuf0, pbuf1,
             sx0, sx1, sx2, sp0, sp1, so0, so1, so2):
        wid = lax.axis_index("c") * NS + lax.axis_index("s")
        row_base = wid * rows_per_w
        xbufs = (xbuf0, xbuf1, xbuf2)
        pbufs = (pbuf0, pbuf1)
        sxs = (sx0, sx1, sx2)
        sps = (sp0, sp1)
        sos = (so0, so1, so2)

        def x_src(c, b):
            return x_hbm.at[b, pl.ds(row_base + c * CHUNK, CHUNK), :]

        def o_dst(c, b):
            return o_hbm.at[b, pl.ds(row_base + c * CHUNK, CHUNK), :]

        def p_src(c):
            return p_hbm.at[pl.ds(row_base + c * CHUNK, CHUNK), :]

        items = [(c, b) for c in range(n_chunks) for b in range(batch)]
        n = len(items)

        pltpu.make_async_copy(p_src(0), pbuf0, sp0).start()
        pltpu.make_async_copy(x_src(0, 0), xbuf0, sx0).start()

        for i, (c, b) in enumerate(items):
            s = i % NB
            ps = c % 2
            if b == 0:
                pltpu.make_async_copy(p_src(c), pbufs[ps], sps[ps]).wait()
                if c + 1 < n_chunks:
                    nps = (c + 1) % 2
                    pltpu.make_async_copy(
                        p_src(c + 1), pbufs[nps], sps[nps]).start()
            if i + 1 < n:
                ns = (i + 1) % NB
                if i >= NB - 1:
                    pc, pb = items[i + 1 - NB]
                    pltpu.make_async_copy(
                        xbufs[ns], o_dst(pc, pb), sos[ns]).wait()
                nc, nb = items[i + 1]
                pltpu.make_async_copy(x_src(nc, nb), xbufs[ns], sxs[ns]).start()
            pltpu.make_async_copy(x_src(c, b), xbufs[s], sxs[s]).wait()

            xb, pb_ = xbufs[s], pbufs[ps]

            def vbody(j, xb=xb, pb_=pb_):
                for r in range(CHUNK):
                    plsc.addupdate(xb.at[r, pl.ds(j, 16)], pb_[r, pl.ds(j, 16)])

            plsc.parallel_loop(0, dm, step=16, unroll=2)(vbody)

            pltpu.make_async_copy(xbufs[s], o_dst(c, b), sos[s]).start()

        for i in range(n - NB, n):
            ce, be = items[i]
            pltpu.make_async_copy(
                xbufs[i % NB], o_dst(ce, be), sos[i % NB]).wait()

    return body


def kernel(inputs, pos_table):
    batch, seq, dm = inputs.shape
    mesh = plsc.VectorSubcoreMesh(core_axis_name="c", subcore_axis_name="s")
    k = functools.partial(
        pl.kernel,
        mesh=mesh,
        out_type=jax.ShapeDtypeStruct((batch, seq, dm), inputs.dtype),
        scratch_types=[
            pltpu.VMEM((CHUNK, dm), jnp.float32),
            pltpu.VMEM((CHUNK, dm), jnp.float32),
            pltpu.VMEM((CHUNK, dm), jnp.float32),
            pltpu.VMEM((CHUNK, dm), jnp.float32),
            pltpu.VMEM((CHUNK, dm), jnp.float32),
            pltpu.SemaphoreType.DMA,
            pltpu.SemaphoreType.DMA,
            pltpu.SemaphoreType.DMA,
            pltpu.SemaphoreType.DMA,
            pltpu.SemaphoreType.DMA,
            pltpu.SemaphoreType.DMA,
            pltpu.SemaphoreType.DMA,
            pltpu.SemaphoreType.DMA,
        ],
    )(_make_body(batch, seq, dm))
    return k(inputs, pos_table)


# speedup vs baseline: 1.1933x; 1.0866x over previous
import functools
import jax
import jax.numpy as jnp
from jax import lax
from jax.experimental import pallas as pl
from jax.experimental.pallas import tpu as pltpu
from jax.experimental.pallas import tpu_sc as plsc

NC = 2
NS = 16
NB = 3
CHUNK = 8
SC_BATCH = 3


def _make_sc_body(seq, dm):
    nw = NC * NS
    rows_per_w = seq // nw
    n_chunks = rows_per_w // CHUNK

    def body(x_hbm, p_hbm, o_hbm,
             xbuf0, xbuf1, xbuf2, pbuf0, pbuf1,
             sx0, sx1, sx2, sp0, sp1, so0, so1, so2):
        wid = lax.axis_index("c") * NS + lax.axis_index("s")
        row_base = wid * rows_per_w
        xbufs = (xbuf0, xbuf1, xbuf2)
        pbufs = (pbuf0, pbuf1)
        sxs = (sx0, sx1, sx2)
        sps = (sp0, sp1)
        sos = (so0, so1, so2)

        def x_src(c):
            return x_hbm.at[SC_BATCH, pl.ds(row_base + c * CHUNK, CHUNK), :]

        def o_dst(c):
            return o_hbm.at[SC_BATCH, pl.ds(row_base + c * CHUNK, CHUNK), :]

        def p_src(c):
            return p_hbm.at[pl.ds(row_base + c * CHUNK, CHUNK), :]

        n = n_chunks

        pltpu.make_async_copy(p_src(0), pbuf0, sp0).start()
        pltpu.make_async_copy(x_src(0), xbuf0, sx0).start()

        for i in range(n):
            s = i % NB
            ps = i % 2
            pltpu.make_async_copy(p_src(i), pbufs[ps], sps[ps]).wait()
            if i + 1 < n:
                nps = (i + 1) % 2
                pltpu.make_async_copy(p_src(i + 1), pbufs[nps], sps[nps]).start()
                ns = (i + 1) % NB
                if i >= NB - 1:
                    pltpu.make_async_copy(
                        xbufs[ns], o_dst(i + 1 - NB), sos[ns]).wait()
                pltpu.make_async_copy(x_src(i + 1), xbufs[ns], sxs[ns]).start()
            pltpu.make_async_copy(x_src(i), xbufs[s], sxs[s]).wait()

            xb, pb_ = xbufs[s], pbufs[ps]

            def vbody(j, xb=xb, pb_=pb_):
                for r in range(CHUNK):
                    plsc.addupdate(xb.at[r, pl.ds(j, 16)], pb_[r, pl.ds(j, 16)])

            plsc.parallel_loop(0, dm, step=16, unroll=2)(vbody)

            pltpu.make_async_copy(xbufs[s], o_dst(i), sos[s]).start()

        for i in range(max(0, n - NB), n):
            pltpu.make_async_copy(xbufs[i % NB], o_dst(i), sos[i % NB]).wait()

    return body


def _sc_batch3(inputs, pos_table):
    batch, seq, dm = inputs.shape
    mesh = plsc.VectorSubcoreMesh(core_axis_name="c", subcore_axis_name="s")
    k = functools.partial(
        pl.kernel,
        mesh=mesh,
        out_type=jax.ShapeDtypeStruct((batch, seq, dm), inputs.dtype),
        scratch_types=[
            pltpu.VMEM((CHUNK, dm), jnp.float32),
            pltpu.VMEM((CHUNK, dm), jnp.float32),
            pltpu.VMEM((CHUNK, dm), jnp.float32),
            pltpu.VMEM((CHUNK, dm), jnp.float32),
            pltpu.VMEM((CHUNK, dm), jnp.float32),
            pltpu.SemaphoreType.DMA,
            pltpu.SemaphoreType.DMA,
            pltpu.SemaphoreType.DMA,
            pltpu.SemaphoreType.DMA,
            pltpu.SemaphoreType.DMA,
            pltpu.SemaphoreType.DMA,
            pltpu.SemaphoreType.DMA,
            pltpu.SemaphoreType.DMA,
        ],
    )(_make_sc_body(seq, dm))
    return k(inputs, pos_table)


def _tc_kernel(x_ref, p_ref, acc_ref, o_ref):
    del acc_ref
    o_ref[...] = x_ref[...] + p_ref[...][None, :, :]


def kernel(inputs, pos_table):
    batch, seq, dm = inputs.shape
    sc_out = _sc_batch3(inputs, pos_table)
    blk_s = 1024
    grid = (seq // blk_s, batch - 1)
    return pl.pallas_call(
        _tc_kernel,
        grid=grid,
        in_specs=[
            pl.BlockSpec((1, blk_s, dm), lambda i, j: (j, i, 0)),
            pl.BlockSpec((blk_s, dm), lambda i, j: (i, 0)),
            pl.BlockSpec(memory_space=pl.ANY),
        ],
        out_specs=pl.BlockSpec((1, blk_s, dm), lambda i, j: (j, i, 0)),
        out_shape=jax.ShapeDtypeStruct(inputs.shape, inputs.dtype),
        input_output_aliases={2: 0},
        compiler_params=pltpu.CompilerParams(vmem_limit_bytes=60 * 1024 * 1024),
    )(inputs, pos_table, sc_out)
